# Initial kernel scaffold; baseline (speedup 1.0000x reference)
import jax
import jax.numpy as jnp
from jax.experimental import pallas as pl


def kernel(entity_type, entity_params, entity_embed_w, param_fc_w, param_fc_b, pos_embed_w):
    raise NotImplementedError("write your pallas kernel here")


# speedup vs baseline: 6.0514x; 6.0514x over previous
import functools

import jax
import jax.numpy as jnp
from jax.experimental import pallas as pl

NUM_TYPES = 13
NUM_PARAMS = 45
D_MODEL = 128
TILE_B = 32


def _seq_embed_kernel(type_ref, params_ref, embed_ref, w_ref, b_ref, pos_ref, out_ref):
    t = type_ref[...]
    safe_t = jnp.where(t < 0, NUM_TYPES - 1, jnp.minimum(t, NUM_TYPES - 1))
    p = params_ref[...]
    p = jnp.where(p < 0, 0.0, p)
    proj = jax.lax.dot_general(
        p, w_ref[...],
        dimension_numbers=(((2,), (0,)), ((), ())),
        preferred_element_type=jnp.float32,
    )
    iota_t = jax.lax.broadcasted_iota(jnp.int32, safe_t.shape + (NUM_TYPES,), 2)
    onehot = (safe_t[:, :, None] == iota_t).astype(jnp.float32)
    emb = jax.lax.dot_general(
        onehot, embed_ref[...],
        dimension_numbers=(((2,), (0,)), ((), ())),
        preferred_element_type=jnp.float32,
    )
    out_ref[...] = proj + emb + b_ref[...][None] + pos_ref[...][None]


def kernel(entity_type, entity_params, entity_embed_w, param_fc_w, param_fc_b, pos_embed_w):
    B, L = entity_type.shape
    P = entity_params.shape[-1]
    D = param_fc_w.shape[-1]
    T = entity_embed_w.shape[0]
    bias2d = param_fc_b.reshape(1, D)
    pos = pos_embed_w[:L]
    grid = (B // TILE_B,)
    return pl.pallas_call(
        _seq_embed_kernel,
        grid=grid,
        in_specs=[
            pl.BlockSpec((TILE_B, L), lambda i: (i, 0)),
            pl.BlockSpec((TILE_B, L, P), lambda i: (i, 0, 0)),
            pl.BlockSpec((T, D), lambda i: (0, 0)),
            pl.BlockSpec((P, D), lambda i: (0, 0)),
            pl.BlockSpec((1, D), lambda i: (0, 0)),
            pl.BlockSpec((L, D), lambda i: (0, 0)),
        ],
        out_specs=pl.BlockSpec((TILE_B, L, D), lambda i: (i, 0, 0)),
        out_shape=jax.ShapeDtypeStruct((B, L, D), jnp.float32),
    )(entity_type, entity_params, entity_embed_w, param_fc_w, bias2d, pos)


# speedup vs baseline: 7.4331x; 1.2283x over previous
import jax
import jax.numpy as jnp
from jax.experimental import pallas as pl

NUM_TYPES = 13


def _seq_embed_kernel(tt_ref, px_ref, embed_ref, w_ref, pb_ref, out_ref):
    lo = pl.program_id(0) % 8
    x = px_ref[:, pl.ds(lo, 1), :].reshape(px_ref.shape[0], px_ref.shape[2])
    x = jnp.where(x < 0, 0.0, x)
    z = jax.lax.dot_general(
        x, w_ref[...],
        dimension_numbers=(((0,), (0,)), ((), ())),
        preferred_element_type=jnp.float32,
    )
    t = tt_ref[pl.ds(lo, 1), :]
    safe_t = jnp.where(t < 0, NUM_TYPES - 1, jnp.minimum(t, NUM_TYPES - 1))
    iota_t = jax.lax.broadcasted_iota(jnp.int32, (NUM_TYPES,) + t.shape[1:], 0)
    onehot = (iota_t == safe_t).astype(jnp.float32)
    emb = jax.lax.dot_general(
        onehot, embed_ref[...],
        dimension_numbers=(((0,), (0,)), ((), ())),
        preferred_element_type=jnp.float32,
    )
    out_ref[:, 0, 0, :] = z + emb + pb_ref[0]


def kernel(entity_type, entity_params, entity_embed_w, param_fc_w, param_fc_b, pos_embed_w):
    B, L = entity_type.shape
    P = entity_params.shape[-1]
    D = param_fc_w.shape[-1]
    T = entity_embed_w.shape[0]
    params_t = jnp.transpose(entity_params, (2, 1, 0))
    type_t = jnp.transpose(entity_type, (1, 0))
    pos_bias = (pos_embed_w[:L] + param_fc_b[None, :]).reshape(L, 1, D)
    out = pl.pallas_call(
        _seq_embed_kernel,
        grid=(L,),
        in_specs=[
            pl.BlockSpec((8, B), lambda l: (l // 8, 0)),
            pl.BlockSpec((P, 8, B), lambda l: (0, l // 8, 0)),
            pl.BlockSpec((T, D), lambda l: (0, 0)),
            pl.BlockSpec((P, D), lambda l: (0, 0)),
            pl.BlockSpec((1, 1, D), lambda l: (l, 0, 0)),
        ],
        out_specs=pl.BlockSpec((B, 1, 1, D), lambda l: (0, l, 0, 0)),
        out_shape=jax.ShapeDtypeStruct((B, L, 1, D), jnp.float32),
    )(type_t, params_t, entity_embed_w, param_fc_w, pos_bias)
    return out.reshape(B, L, D)


# speedup vs baseline: 11.0000x; 1.4799x over previous
import jax
import jax.numpy as jnp
from jax.experimental import pallas as pl
from jax.experimental.pallas import tpu as pltpu

NUM_TYPES = 13


def _seq_embed_kernel(tt_ref, px_ref, rhs_ref, out_hbm, scratch, sem):
    l = pl.program_id(0)
    n = pl.num_programs(0)
    lo = l % 8
    slot = l % 2
    P, B = px_ref.shape[0], px_ref.shape[2]

    @pl.when(l >= 2)
    def _wait_prev():
        pltpu.make_async_copy(
            scratch.at[slot], out_hbm.at[:, l - 2, :], sem.at[slot]
        ).wait()

    x = px_ref[:, pl.ds(lo, 1), :].reshape(P, B)
    x = jnp.maximum(x, 0.0)
    t = tt_ref[pl.ds(lo, 1), :]
    safe_t = jnp.where(t < 0, NUM_TYPES - 1, jnp.minimum(t, NUM_TYPES - 1))
    iota_t = jax.lax.broadcasted_iota(jnp.int32, (NUM_TYPES, B), 0)
    onehot = (iota_t == safe_t).astype(jnp.float32)
    ones = jnp.ones((1, B), jnp.float32)
    lhs = jnp.concatenate([x, onehot, ones], axis=0)
    y = jax.lax.dot_general(
        lhs, rhs_ref[0],
        dimension_numbers=(((0,), (0,)), ((), ())),
        preferred_element_type=jnp.float32,
        precision=jax.lax.Precision.DEFAULT,
    )
    scratch[slot] = y
    pltpu.make_async_copy(
        scratch.at[slot], out_hbm.at[:, l, :], sem.at[slot]
    ).start()

    @pl.when(l == n - 1)
    def _drain():
        pltpu.make_async_copy(
            scratch.at[slot], out_hbm.at[:, l, :], sem.at[slot]
        ).wait()
        pltpu.make_async_copy(
            scratch.at[1 - slot], out_hbm.at[:, l - 1, :], sem.at[1 - slot]
        ).wait()


def kernel(entity_type, entity_params, entity_embed_w, param_fc_w, param_fc_b, pos_embed_w):
    B, L = entity_type.shape
    P = entity_params.shape[-1]
    D = param_fc_w.shape[-1]
    T = entity_embed_w.shape[0]
    params_t = jnp.transpose(entity_params, (2, 1, 0))
    type_t = jnp.transpose(entity_type, (1, 0))
    rhs = jnp.concatenate(
        [
            jnp.broadcast_to(param_fc_w[None], (L, P, D)),
            jnp.broadcast_to(entity_embed_w[None], (L, T, D)),
            (pos_embed_w[:L] + param_fc_b[None, :])[:, None, :],
        ],
        axis=1,
    )
    return pl.pallas_call(
        _seq_embed_kernel,
        grid=(L,),
        in_specs=[
            pl.BlockSpec((8, B), lambda l: (l // 8, 0)),
            pl.BlockSpec((P, 8, B), lambda l: (0, l // 8, 0)),
            pl.BlockSpec((1, P + T + 1, D), lambda l: (l, 0, 0)),
        ],
        out_specs=pl.BlockSpec(memory_space=pl.ANY),
        out_shape=jax.ShapeDtypeStruct((B, L, D), jnp.float32),
        scratch_shapes=[
            pltpu.VMEM((2, B, D), jnp.float32),
            pltpu.SemaphoreType.DMA((2,)),
        ],
    )(type_t, params_t, rhs)


# speedup vs baseline: 12.1754x; 1.1069x over previous
import jax
import jax.numpy as jnp
from jax.experimental import pallas as pl
from jax.experimental.pallas import tpu as pltpu

NUM_TYPES = 13
NBUF = 4


def _seq_embed_kernel(tt_ref, px_ref, rhs_ref, out_hbm, scratch, sem):
    l = pl.program_id(0)
    n = pl.num_programs(0)
    lo = l % 8
    slot = l % NBUF
    P, B = px_ref.shape[0], px_ref.shape[2]

    @pl.when(l >= NBUF)
    def _wait_prev():
        pltpu.make_async_copy(
            scratch.at[slot], out_hbm.at[:, l - NBUF, :], sem.at[slot]
        ).wait()

    x = px_ref[:, pl.ds(lo, 1), :].reshape(P, B)
    x = jnp.maximum(x, 0.0)
    t = tt_ref[pl.ds(lo, 1), :]
    safe_t = jnp.where(t < 0, NUM_TYPES - 1, jnp.minimum(t, NUM_TYPES - 1))
    iota_t = jax.lax.broadcasted_iota(jnp.int32, (NUM_TYPES, B), 0)
    onehot = (iota_t == safe_t).astype(jnp.float32)
    ones = jnp.ones((1, B), jnp.float32)
    lhs = jnp.concatenate([x, onehot, ones], axis=0)
    y = jax.lax.dot_general(
        lhs, rhs_ref[0],
        dimension_numbers=(((0,), (0,)), ((), ())),
        preferred_element_type=jnp.float32,
        precision=jax.lax.Precision.DEFAULT,
    )
    scratch[slot] = y
    pltpu.make_async_copy(
        scratch.at[slot], out_hbm.at[:, l, :], sem.at[slot]
    ).start()

    @pl.when(l == n - 1)
    def _drain():
        for k in range(NBUF):
            pltpu.make_async_copy(
                scratch.at[(slot - k) % NBUF], out_hbm.at[:, l - k, :],
                sem.at[(slot - k) % NBUF]
            ).wait()


def kernel(entity_type, entity_params, entity_embed_w, param_fc_w, param_fc_b, pos_embed_w):
    B, L = entity_type.shape
    P = entity_params.shape[-1]
    D = param_fc_w.shape[-1]
    T = entity_embed_w.shape[0]
    params_t = jnp.transpose(entity_params, (2, 1, 0))
    type_t = jnp.transpose(entity_type, (1, 0))
    rhs = jnp.concatenate(
        [
            jnp.broadcast_to(param_fc_w[None], (L, P, D)),
            jnp.broadcast_to(entity_embed_w[None], (L, T, D)),
            (pos_embed_w[:L] + param_fc_b[None, :])[:, None, :],
        ],
        axis=1,
    )
    return pl.pallas_call(
        _seq_embed_kernel,
        grid=(L,),
        in_specs=[
            pl.BlockSpec((8, B), lambda l: (l // 8, 0)),
            pl.BlockSpec((P, 8, B), lambda l: (0, l // 8, 0)),
            pl.BlockSpec((1, P + T + 1, D), lambda l: (l, 0, 0)),
        ],
        out_specs=pl.BlockSpec(memory_space=pl.ANY),
        out_shape=jax.ShapeDtypeStruct((B, L, D), jnp.float32),
        scratch_shapes=[
            pltpu.VMEM((NBUF, B, D), jnp.float32),
            pltpu.SemaphoreType.DMA((NBUF,)),
        ],
    )(type_t, params_t, rhs)


# speedup vs baseline: 15.4500x; 1.2689x over previous
import jax
import jax.numpy as jnp
from jax.experimental import pallas as pl
from jax.experimental.pallas import tpu as pltpu

NUM_TYPES = 13
NIB = 3
NOB = 6


def _seq_embed_kernel(tt_ref, rhs_ref, px_hbm, out_hbm, px2, scratch, isem, osem):
    l = pl.program_id(0)
    n = pl.num_programs(0)
    lo = l % 8
    islot = l % NIB
    oslot = l % NOB
    P = px_hbm.shape[0]
    B = px_hbm.shape[2]

    @pl.when(l == 0)
    def _prologue():
        pltpu.make_async_copy(px_hbm.at[:, 0, :], px2.at[0], isem.at[0]).start()
        pltpu.make_async_copy(px_hbm.at[:, 1, :], px2.at[1], isem.at[1]).start()

    @pl.when(l + 2 < n)
    def _prefetch():
        s = (l + 2) % NIB
        pltpu.make_async_copy(px_hbm.at[:, l + 2, :], px2.at[s], isem.at[s]).start()

    @pl.when(l >= NOB)
    def _wait_out():
        pltpu.make_async_copy(
            scratch.at[oslot], out_hbm.at[:, l - NOB, :], osem.at[oslot]
        ).wait()

    pltpu.make_async_copy(px_hbm.at[:, l, :], px2.at[islot], isem.at[islot]).wait()

    x = jnp.maximum(px2[islot], 0.0)
    t = tt_ref[pl.ds(lo, 1), :]
    safe_t = jnp.where(t < 0, NUM_TYPES - 1, jnp.minimum(t, NUM_TYPES - 1))
    iota_t = jax.lax.broadcasted_iota(jnp.int32, (NUM_TYPES, B), 0)
    onehot = (iota_t == safe_t).astype(jnp.float32)
    ones = jnp.ones((1, B), jnp.float32)
    lhs = jnp.concatenate([x, onehot, ones], axis=0)
    y = jax.lax.dot_general(
        lhs, rhs_ref[0],
        dimension_numbers=(((0,), (0,)), ((), ())),
        preferred_element_type=jnp.float32,
        precision=jax.lax.Precision.DEFAULT,
    )
    scratch[oslot] = y
    pltpu.make_async_copy(
        scratch.at[oslot], out_hbm.at[:, l, :], osem.at[oslot]
    ).start()

    @pl.when(l == n - 1)
    def _drain():
        for k in range(NOB):
            pltpu.make_async_copy(
                scratch.at[(oslot - k) % NOB], out_hbm.at[:, l - k, :],
                osem.at[(oslot - k) % NOB]
            ).wait()


def kernel(entity_type, entity_params, entity_embed_w, param_fc_w, param_fc_b, pos_embed_w):
    B, L = entity_type.shape
    P = entity_params.shape[-1]
    D = param_fc_w.shape[-1]
    T = entity_embed_w.shape[0]
    params_t = jnp.transpose(entity_params, (2, 1, 0))
    type_t = jnp.transpose(entity_type, (1, 0))
    rhs = jnp.concatenate(
        [
            jnp.broadcast_to(param_fc_w[None], (L, P, D)),
            jnp.broadcast_to(entity_embed_w[None], (L, T, D)),
            (pos_embed_w[:L] + param_fc_b[None, :])[:, None, :],
        ],
        axis=1,
    )
    return pl.pallas_call(
        _seq_embed_kernel,
        grid=(L,),
        in_specs=[
            pl.BlockSpec((8, B), lambda l: (l // 8, 0)),
            pl.BlockSpec((1, P + T + 1, D), lambda l: (l, 0, 0)),
            pl.BlockSpec(memory_space=pl.ANY),
        ],
        out_specs=pl.BlockSpec(memory_space=pl.ANY),
        out_shape=jax.ShapeDtypeStruct((B, L, D), jnp.float32),
        scratch_shapes=[
            pltpu.VMEM((NIB, P, B), jnp.float32),
            pltpu.VMEM((NOB, B, D), jnp.float32),
            pltpu.SemaphoreType.DMA((NIB,)),
            pltpu.SemaphoreType.DMA((NOB,)),
        ],
    )(type_t, rhs, params_t)


# speedup vs baseline: 15.4813x; 1.0020x over previous
import jax
import jax.numpy as jnp
from jax.experimental import pallas as pl
from jax.experimental.pallas import tpu as pltpu

NUM_TYPES = 13
NIB = 4
NOB = 8


def _seq_embed_kernel(tt_ref, rhs_ref, px_hbm, out_hbm, px2, scratch, isem, osem):
    l = pl.program_id(0)
    n = pl.num_programs(0)
    lo = l % 8
    islot = l % NIB
    oslot = l % NOB
    P = px_hbm.shape[0]
    B = px_hbm.shape[2]

    @pl.when(l == 0)
    def _prologue():
        pltpu.make_async_copy(px_hbm.at[:, 0, :], px2.at[0], isem.at[0]).start()
        pltpu.make_async_copy(px_hbm.at[:, 1, :], px2.at[1], isem.at[1]).start()

    @pl.when(l + 2 < n)
    def _prefetch():
        s = (l + 2) % NIB
        pltpu.make_async_copy(px_hbm.at[:, l + 2, :], px2.at[s], isem.at[s]).start()

    @pl.when(l >= NOB)
    def _wait_out():
        pltpu.make_async_copy(
            scratch.at[oslot], out_hbm.at[:, l - NOB, :], osem.at[oslot]
        ).wait()

    pltpu.make_async_copy(px_hbm.at[:, l, :], px2.at[islot], isem.at[islot]).wait()

    x = jnp.maximum(px2[islot], 0.0)
    t = tt_ref[pl.ds(lo, 1), :]
    safe_t = jnp.where(t < 0, NUM_TYPES - 1, jnp.minimum(t, NUM_TYPES - 1))
    iota_t = jax.lax.broadcasted_iota(jnp.int32, (NUM_TYPES, B), 0)
    onehot = (iota_t == safe_t).astype(jnp.float32)
    ones = jnp.ones((1, B), jnp.float32)
    lhs = jnp.concatenate([x, onehot, ones], axis=0)
    y = jax.lax.dot_general(
        lhs, rhs_ref[0],
        dimension_numbers=(((0,), (0,)), ((), ())),
        preferred_element_type=jnp.float32,
        precision=jax.lax.Precision.DEFAULT,
    )
    scratch[oslot] = y
    pltpu.make_async_copy(
        scratch.at[oslot], out_hbm.at[:, l, :], osem.at[oslot]
    ).start()

    @pl.when(l == n - 1)
    def _drain():
        for k in range(NOB):
            pltpu.make_async_copy(
                scratch.at[(oslot - k) % NOB], out_hbm.at[:, l - k, :],
                osem.at[(oslot - k) % NOB]
            ).wait()


def kernel(entity_type, entity_params, entity_embed_w, param_fc_w, param_fc_b, pos_embed_w):
    B, L = entity_type.shape
    P = entity_params.shape[-1]
    D = param_fc_w.shape[-1]
    T = entity_embed_w.shape[0]
    params_t = jnp.transpose(entity_params, (2, 1, 0))
    type_t = jnp.transpose(entity_type, (1, 0))
    rhs = jnp.concatenate(
        [
            jnp.broadcast_to(param_fc_w[None], (L, P, D)),
            jnp.broadcast_to(entity_embed_w[None], (L, T, D)),
            (pos_embed_w[:L] + param_fc_b[None, :])[:, None, :],
        ],
        axis=1,
    )
    return pl.pallas_call(
        _seq_embed_kernel,
        grid=(L,),
        in_specs=[
            pl.BlockSpec((8, B), lambda l: (l // 8, 0)),
            pl.BlockSpec((1, P + T + 1, D), lambda l: (l, 0, 0)),
            pl.BlockSpec(memory_space=pl.ANY),
        ],
        out_specs=pl.BlockSpec(memory_space=pl.ANY),
        out_shape=jax.ShapeDtypeStruct((B, L, D), jnp.float32),
        scratch_shapes=[
            pltpu.VMEM((NIB, P, B), jnp.float32),
            pltpu.VMEM((NOB, B, D), jnp.float32),
            pltpu.SemaphoreType.DMA((NIB,)),
            pltpu.SemaphoreType.DMA((NOB,)),
        ],
    )(type_t, rhs, params_t)


# speedup vs baseline: 17.7483x; 1.1464x over previous
import jax
import jax.numpy as jnp
from jax.experimental import pallas as pl
from jax.experimental.pallas import tpu as pltpu

NUM_TYPES = 13
LPS = 2
NIB = 6
NOB = 8


def _in_copy(px_hbm, px2, isem, j):
    return pltpu.make_async_copy(
        px_hbm.at[:, j, :], px2.at[j % NIB], isem.at[j % NIB]
    )


def _out_copy(scratch, out_hbm, osem, j):
    return pltpu.make_async_copy(
        scratch.at[j % NOB], out_hbm.at[:, j, :], osem.at[j % NOB]
    )


def _seq_embed_kernel(tt_ref, rhs_ref, px_hbm, out_hbm, px2, scratch, isem, osem):
    s = pl.program_id(0)
    ns = pl.num_programs(0)
    l0 = s * LPS
    n = ns * LPS
    B = px_hbm.shape[2]

    @pl.when(s == 0)
    def _prologue():
        for j in range(2 * LPS):
            _in_copy(px_hbm, px2, isem, j).start()

    for i in range(LPS):
        jpf = l0 + i + 2 * LPS

        @pl.when(jpf < n)
        def _prefetch(jpf=jpf):
            _in_copy(px_hbm, px2, isem, jpf).start()

    for i in range(LPS):
        l = l0 + i
        lo = l % 8

        @pl.when(l >= NOB)
        def _wait_out(l=l):
            _out_copy(scratch, out_hbm, osem, l - NOB).wait()

        _in_copy(px_hbm, px2, isem, l).wait()
        x = jnp.maximum(px2[l % NIB], 0.0)
        t = tt_ref[pl.ds(lo, 1), :]
        safe_t = jnp.where(t < 0, NUM_TYPES - 1, jnp.minimum(t, NUM_TYPES - 1))
        iota_t = jax.lax.broadcasted_iota(jnp.int32, (NUM_TYPES, B), 0)
        onehot = (iota_t == safe_t).astype(jnp.float32)
        ones = jnp.ones((1, B), jnp.float32)
        lhs = jnp.concatenate([x, onehot, ones], axis=0)
        y = jax.lax.dot_general(
            lhs, rhs_ref[i],
            dimension_numbers=(((0,), (0,)), ((), ())),
            preferred_element_type=jnp.float32,
            precision=jax.lax.Precision.DEFAULT,
        )
        scratch[l % NOB] = y
        _out_copy(scratch, out_hbm, osem, l).start()

    @pl.when(s == ns - 1)
    def _drain():
        for k in range(NOB):
            _out_copy(scratch, out_hbm, osem, n - 1 - k).wait()


def kernel(entity_type, entity_params, entity_embed_w, param_fc_w, param_fc_b, pos_embed_w):
    B, L = entity_type.shape
    P = entity_params.shape[-1]
    D = param_fc_w.shape[-1]
    T = entity_embed_w.shape[0]
    params_t = jnp.transpose(entity_params, (2, 1, 0))
    type_t = jnp.transpose(entity_type, (1, 0))
    rhs = jnp.concatenate(
        [
            jnp.broadcast_to(param_fc_w[None], (L, P, D)),
            jnp.broadcast_to(entity_embed_w[None], (L, T, D)),
            (pos_embed_w[:L] + param_fc_b[None, :])[:, None, :],
        ],
        axis=1,
    )
    return pl.pallas_call(
        _seq_embed_kernel,
        grid=(L // LPS,),
        in_specs=[
            pl.BlockSpec((8, B), lambda s: (s * LPS // 8, 0)),
            pl.BlockSpec((LPS, P + T + 1, D), lambda s: (s, 0, 0)),
            pl.BlockSpec(memory_space=pl.ANY),
        ],
        out_specs=pl.BlockSpec(memory_space=pl.ANY),
        out_shape=jax.ShapeDtypeStruct((B, L, D), jnp.float32),
        scratch_shapes=[
            pltpu.VMEM((NIB, P, B), jnp.float32),
            pltpu.VMEM((NOB, B, D), jnp.float32),
            pltpu.SemaphoreType.DMA((NIB,)),
            pltpu.SemaphoreType.DMA((NOB,)),
        ],
    )(type_t, rhs, params_t)


# speedup vs baseline: 18.3317x; 1.0329x over previous
import jax
import jax.numpy as jnp
from jax.experimental import pallas as pl
from jax.experimental.pallas import tpu as pltpu

NUM_TYPES = 13
LPS = 4
NIB = 12
NOB = 8


def _in_copy(px_hbm, px2, isem, j):
    return pltpu.make_async_copy(
        px_hbm.at[:, j, :], px2.at[j % NIB], isem.at[j % NIB]
    )


def _out_copy(scratch, out_hbm, osem, j):
    return pltpu.make_async_copy(
        scratch.at[j % NOB], out_hbm.at[:, j, :], osem.at[j % NOB]
    )


def _seq_embed_kernel(tt_ref, rhs_ref, px_hbm, out_hbm, px2, scratch, isem, osem):
    s = pl.program_id(0)
    ns = pl.num_programs(0)
    l0 = s * LPS
    n = ns * LPS
    B = px_hbm.shape[2]

    @pl.when(s == 0)
    def _prologue():
        for j in range(2 * LPS):
            _in_copy(px_hbm, px2, isem, j).start()

    for i in range(LPS):
        jpf = l0 + i + 2 * LPS

        @pl.when(jpf < n)
        def _prefetch(jpf=jpf):
            _in_copy(px_hbm, px2, isem, jpf).start()

    for i in range(LPS):
        l = l0 + i
        lo = l % 8

        @pl.when(l >= NOB)
        def _wait_out(l=l):
            _out_copy(scratch, out_hbm, osem, l - NOB).wait()

        _in_copy(px_hbm, px2, isem, l).wait()
        x = jnp.maximum(px2[l % NIB], 0.0)
        t = tt_ref[pl.ds(lo, 1), :]
        safe_t = jnp.where(t < 0, NUM_TYPES - 1, jnp.minimum(t, NUM_TYPES - 1))
        iota_t = jax.lax.broadcasted_iota(jnp.int32, (NUM_TYPES, B), 0)
        onehot = (iota_t == safe_t).astype(jnp.float32)
        ones = jnp.ones((1, B), jnp.float32)
        lhs = jnp.concatenate([x, onehot, ones], axis=0)
        y = jax.lax.dot_general(
            lhs, rhs_ref[i],
            dimension_numbers=(((0,), (0,)), ((), ())),
            preferred_element_type=jnp.float32,
            precision=jax.lax.Precision.DEFAULT,
        )
        scratch[l % NOB] = y
        _out_copy(scratch, out_hbm, osem, l).start()

    @pl.when(s == ns - 1)
    def _drain():
        for k in range(NOB):
            _out_copy(scratch, out_hbm, osem, n - 1 - k).wait()


def kernel(entity_type, entity_params, entity_embed_w, param_fc_w, param_fc_b, pos_embed_w):
    B, L = entity_type.shape
    P = entity_params.shape[-1]
    D = param_fc_w.shape[-1]
    T = entity_embed_w.shape[0]
    params_t = jnp.transpose(entity_params, (2, 1, 0))
    type_t = jnp.transpose(entity_type, (1, 0))
    rhs = jnp.concatenate(
        [
            jnp.broadcast_to(param_fc_w[None], (L, P, D)),
            jnp.broadcast_to(entity_embed_w[None], (L, T, D)),
            (pos_embed_w[:L] + param_fc_b[None, :])[:, None, :],
        ],
        axis=1,
    )
    return pl.pallas_call(
        _seq_embed_kernel,
        grid=(L // LPS,),
        in_specs=[
            pl.BlockSpec((8, B), lambda s: (s * LPS // 8, 0)),
            pl.BlockSpec((LPS, P + T + 1, D), lambda s: (s, 0, 0)),
            pl.BlockSpec(memory_space=pl.ANY),
        ],
        out_specs=pl.BlockSpec(memory_space=pl.ANY),
        out_shape=jax.ShapeDtypeStruct((B, L, D), jnp.float32),
        scratch_shapes=[
            pltpu.VMEM((NIB, P, B), jnp.float32),
            pltpu.VMEM((NOB, B, D), jnp.float32),
            pltpu.SemaphoreType.DMA((NIB,)),
            pltpu.SemaphoreType.DMA((NOB,)),
        ],
    )(type_t, rhs, params_t)


# speedup vs baseline: 18.8265x; 1.0270x over previous
import jax
import jax.numpy as jnp
from jax.experimental import pallas as pl
from jax.experimental.pallas import tpu as pltpu

NUM_TYPES = 13
LPS = 8
NIB = 24
NOB = 16


def _in_copy(px_hbm, px2, isem, j):
    return pltpu.make_async_copy(
        px_hbm.at[:, j, :], px2.at[j % NIB], isem.at[j % NIB]
    )


def _out_copy(scratch, out_hbm, osem, j):
    return pltpu.make_async_copy(
        scratch.at[j % NOB], out_hbm.at[:, j, :], osem.at[j % NOB]
    )


def _seq_embed_kernel(tt_ref, rhs_ref, px_hbm, out_hbm, px2, scratch, isem, osem):
    s = pl.program_id(0)
    ns = pl.num_programs(0)
    l0 = s * LPS
    n = ns * LPS
    B = px_hbm.shape[2]

    @pl.when(s == 0)
    def _prologue():
        for j in range(2 * LPS):
            _in_copy(px_hbm, px2, isem, j).start()

    for i in range(LPS):
        jpf = l0 + i + 2 * LPS

        @pl.when(jpf < n)
        def _prefetch(jpf=jpf):
            _in_copy(px_hbm, px2, isem, jpf).start()

    for i in range(LPS):
        l = l0 + i
        lo = l % 8

        @pl.when(l >= NOB)
        def _wait_out(l=l):
            _out_copy(scratch, out_hbm, osem, l - NOB).wait()

        _in_copy(px_hbm, px2, isem, l).wait()
        x = jnp.maximum(px2[l % NIB], 0.0)
        t = tt_ref[pl.ds(lo, 1), :]
        safe_t = jnp.where(t < 0, NUM_TYPES - 1, jnp.minimum(t, NUM_TYPES - 1))
        iota_t = jax.lax.broadcasted_iota(jnp.int32, (NUM_TYPES, B), 0)
        onehot = (iota_t == safe_t).astype(jnp.float32)
        ones = jnp.ones((1, B), jnp.float32)
        lhs = jnp.concatenate([x, onehot, ones], axis=0)
        y = jax.lax.dot_general(
            lhs, rhs_ref[i],
            dimension_numbers=(((0,), (0,)), ((), ())),
            preferred_element_type=jnp.float32,
            precision=jax.lax.Precision.DEFAULT,
        )
        scratch[l % NOB] = y
        _out_copy(scratch, out_hbm, osem, l).start()

    @pl.when(s == ns - 1)
    def _drain():
        for k in range(NOB):
            _out_copy(scratch, out_hbm, osem, n - 1 - k).wait()


def kernel(entity_type, entity_params, entity_embed_w, param_fc_w, param_fc_b, pos_embed_w):
    B, L = entity_type.shape
    P = entity_params.shape[-1]
    D = param_fc_w.shape[-1]
    T = entity_embed_w.shape[0]
    params_t = jnp.transpose(entity_params, (2, 1, 0))
    type_t = jnp.transpose(entity_type, (1, 0))
    rhs = jnp.concatenate(
        [
            jnp.broadcast_to(param_fc_w[None], (L, P, D)),
            jnp.broadcast_to(entity_embed_w[None], (L, T, D)),
            (pos_embed_w[:L] + param_fc_b[None, :])[:, None, :],
        ],
        axis=1,
    )
    return pl.pallas_call(
        _seq_embed_kernel,
        grid=(L // LPS,),
        in_specs=[
            pl.BlockSpec((8, B), lambda s: (s * LPS // 8, 0)),
            pl.BlockSpec((LPS, P + T + 1, D), lambda s: (s, 0, 0)),
            pl.BlockSpec(memory_space=pl.ANY),
        ],
        out_specs=pl.BlockSpec(memory_space=pl.ANY),
        out_shape=jax.ShapeDtypeStruct((B, L, D), jnp.float32),
        scratch_shapes=[
            pltpu.VMEM((NIB, P, B), jnp.float32),
            pltpu.VMEM((NOB, B, D), jnp.float32),
            pltpu.SemaphoreType.DMA((NIB,)),
            pltpu.SemaphoreType.DMA((NOB,)),
        ],
    )(type_t, rhs, params_t)
